# Initial kernel scaffold; baseline (speedup 1.0000x reference)
import jax
import jax.numpy as jnp
from jax.experimental import pallas as pl


def kernel(x, Wg, We1, We3, We2, Ws1, Ws3, Ws2):
    raise NotImplementedError("write your pallas kernel here")


# speedup vs baseline: 8.9883x; 8.9883x over previous
import functools

import jax
import jax.numpy as jnp
from jax import lax
from jax.experimental import pallas as pl
from jax.experimental.pallas import tpu as pltpu
from jax.experimental.pallas import tpu_sc as plsc

DIM = 768
D_FF = 256
N_EXPERTS = 64
N_SHARED = 2
N_ROUTED = N_EXPERTS - N_SHARED
SHARED_FF = N_SHARED * D_FF
T = 4 * 2048

EP = 64
BT_R = 512
NB_R = T // BT_R
BT_G = 128
NT = (T + N_ROUTED * (BT_G - 1) + BT_G - 1) // BT_G
P = NT * BT_G
BT_S = 512
NB_S = T // BT_S


_SC_INFO = plsc.get_sparse_core_info()
_NC = _SC_INFO.num_cores
_NS = _SC_INFO.num_subcores
_NW = _NC * _NS
_TPW = T // _NW
_CH = 128
_NCH = _TPW // _CH


def _sc_scatter_body(pos_hbm, x_hbm, xs_hbm, idx_v, rows_v, sem):
    wid = lax.axis_index("s") * _NC + lax.axis_index("c")
    base = wid * _TPW
    for c in range(_NCH):
        off = base + c * _CH
        pltpu.sync_copy(pos_hbm.at[pl.ds(off, _CH)], idx_v)
        pltpu.sync_copy(x_hbm.at[pl.ds(off, _CH)], rows_v)
        pltpu.async_copy(rows_v, xs_hbm.at[idx_v], sem).wait()


def _sc_gather_body(pos_hbm, src_hbm, out_hbm, idx_v, rows_v, sem):
    wid = lax.axis_index("s") * _NC + lax.axis_index("c")
    base = wid * _TPW
    for c in range(_NCH):
        off = base + c * _CH
        pltpu.sync_copy(pos_hbm.at[pl.ds(off, _CH)], idx_v)
        pltpu.async_copy(src_hbm.at[idx_v], rows_v, sem).wait()
        pltpu.sync_copy(rows_v, out_hbm.at[pl.ds(off, _CH)])


_sc_scatter = functools.partial(
    pl.kernel,
    mesh=plsc.VectorSubcoreMesh(core_axis_name="c", subcore_axis_name="s"),
    out_type=jax.ShapeDtypeStruct((P, DIM), jnp.float32),
    scratch_types=[
        pltpu.VMEM((_CH,), jnp.int32),
        pltpu.VMEM((_CH, DIM), jnp.float32),
        pltpu.SemaphoreType.DMA,
    ],
)(_sc_scatter_body)

_sc_gather = functools.partial(
    pl.kernel,
    mesh=plsc.VectorSubcoreMesh(core_axis_name="c", subcore_axis_name="s"),
    out_type=jax.ShapeDtypeStruct((T, DIM), jnp.float32),
    scratch_types=[
        pltpu.VMEM((_CH,), jnp.int32),
        pltpu.VMEM((_CH, DIM), jnp.float32),
        pltpu.SemaphoreType.DMA,
    ],
)(_sc_gather_body)


def _router_body(x_ref, wg_ref, eid_ref, gate_ref, rank_ref, counts_ref, cnt):
    pid = pl.program_id(0)

    @pl.when(pid == 0)
    def _():
        cnt[...] = jnp.zeros_like(cnt)

    x = x_ref[...]
    logits = jnp.dot(x, wg_ref[...], preferred_element_type=jnp.float32)
    col = jax.lax.broadcasted_iota(jnp.int32, logits.shape, 1)
    logits = jnp.where(col < N_ROUTED, logits, -1e30)
    m = jnp.max(logits, axis=1, keepdims=True)
    ssum = jnp.sum(jnp.exp(logits - m), axis=1)
    eid = jnp.min(jnp.where(logits == m, col, EP), axis=1)
    onehot = (col == eid[:, None]).astype(jnp.float32)
    row_i = jax.lax.broadcasted_iota(jnp.int32, (BT_R, BT_R), 0)
    col_i = jax.lax.broadcasted_iota(jnp.int32, (BT_R, BT_R), 1)
    tri = (row_i >= col_i).astype(jnp.float32)
    csum = jnp.dot(tri, onehot, preferred_element_type=jnp.float32)
    rank_in_blk = jnp.sum(onehot * csum, axis=1) - 1.0
    prev = jnp.sum(onehot * cnt[0:1, :], axis=1)
    eid_ref[0, 0, :] = eid
    gate_ref[0, 0, :] = 1.0 / ssum
    rank_ref[0, 0, :] = (prev + rank_in_blk).astype(jnp.int32)
    cnt[0:1, :] = cnt[0:1, :] + csum[BT_R - 1:BT_R, :]

    @pl.when(pid == NB_R - 1)
    def _():
        counts_ref[...] = jnp.broadcast_to(cnt[0:1, :], (8, EP)).astype(jnp.int32)


def _moe_body(te_ref, xs_ref, w1_ref, w3_ref, w2_ref, out_ref):
    del te_ref
    x = xs_ref[...]
    w1 = w1_ref[0]
    w3 = w3_ref[0]
    w2 = w2_ref[0]
    dn = (((1,), (1,)), ((), ()))
    a = jax.lax.dot_general(x, w1, dn, preferred_element_type=jnp.float32)
    b = jax.lax.dot_general(x, w3, dn, preferred_element_type=jnp.float32)
    h = a * jax.lax.logistic(a) * b
    out_ref[...] = jax.lax.dot_general(h, w2, dn,
                                       preferred_element_type=jnp.float32)


def _shared_body(x_ref, w1_ref, w3_ref, w2_ref, yg_ref, gw_ref, out_ref):
    x = x_ref[...]
    dn = (((1,), (1,)), ((), ()))
    a = jax.lax.dot_general(x, w1_ref[...], dn,
                            preferred_element_type=jnp.float32)
    b = jax.lax.dot_general(x, w3_ref[...], dn,
                            preferred_element_type=jnp.float32)
    h = a * jax.lax.logistic(a) * b
    z = jax.lax.dot_general(h, w2_ref[...], dn,
                            preferred_element_type=jnp.float32)
    w = gw_ref[0, 0, :]
    out_ref[...] = z + yg_ref[...] * w[:, None]


def kernel(x, Wg, We1, We3, We2, Ws1, Ws3, Ws2):
    shape = x.shape
    xf = x.reshape(T, DIM)
    wg_pad = jnp.pad(Wg, ((0, 0), (0, EP - N_ROUTED)))

    eid3, gate3, rank3, counts = pl.pallas_call(
        _router_body,
        grid=(NB_R,),
        in_specs=[
            pl.BlockSpec((BT_R, DIM), lambda i: (i, 0)),
            pl.BlockSpec((DIM, EP), lambda i: (0, 0)),
        ],
        out_specs=[
            pl.BlockSpec((1, 1, BT_R), lambda i: (i, 0, 0)),
            pl.BlockSpec((1, 1, BT_R), lambda i: (i, 0, 0)),
            pl.BlockSpec((1, 1, BT_R), lambda i: (i, 0, 0)),
            pl.BlockSpec((8, EP), lambda i: (0, 0)),
        ],
        out_shape=[
            jax.ShapeDtypeStruct((NB_R, 1, BT_R), jnp.int32),
            jax.ShapeDtypeStruct((NB_R, 1, BT_R), jnp.float32),
            jax.ShapeDtypeStruct((NB_R, 1, BT_R), jnp.int32),
            jax.ShapeDtypeStruct((8, EP), jnp.int32),
        ],
        scratch_shapes=[pltpu.VMEM((1, EP), jnp.float32)],
    )(xf, wg_pad)

    eid = eid3.reshape(T)
    rank = rank3.reshape(T)
    c = counts[0, :N_ROUTED]
    padded = ((c + BT_G - 1) // BT_G) * BT_G
    starts = jnp.concatenate([jnp.zeros((1,), jnp.int32),
                              jnp.cumsum(padded)[:-1].astype(jnp.int32)])
    pos = starts[eid] + rank
    tiles_cum = jnp.cumsum(padded // BT_G).astype(jnp.int32)
    tile_expert = jnp.minimum(
        jnp.searchsorted(tiles_cum, jnp.arange(NT, dtype=jnp.int32),
                         side="right").astype(jnp.int32),
        N_ROUTED - 1)

    xs = _sc_scatter(pos, xf)

    out_padded = pl.pallas_call(
        _moe_body,
        grid_spec=pltpu.PrefetchScalarGridSpec(
            num_scalar_prefetch=1,
            grid=(NT,),
            in_specs=[
                pl.BlockSpec((BT_G, DIM), lambda j, te: (j, 0)),
                pl.BlockSpec((1, D_FF, DIM), lambda j, te: (te[j], 0, 0)),
                pl.BlockSpec((1, D_FF, DIM), lambda j, te: (te[j], 0, 0)),
                pl.BlockSpec((1, DIM, D_FF), lambda j, te: (te[j], 0, 0)),
            ],
            out_specs=pl.BlockSpec((BT_G, DIM), lambda j, te: (j, 0)),
        ),
        out_shape=jax.ShapeDtypeStruct((P, DIM), jnp.float32),
    )(tile_expert, xs, We1, We3, We2)

    yg = _sc_gather(pos, out_padded)

    out = pl.pallas_call(
        _shared_body,
        grid=(NB_S,),
        in_specs=[
            pl.BlockSpec((BT_S, DIM), lambda i: (i, 0)),
            pl.BlockSpec((SHARED_FF, DIM), lambda i: (0, 0)),
            pl.BlockSpec((SHARED_FF, DIM), lambda i: (0, 0)),
            pl.BlockSpec((DIM, SHARED_FF), lambda i: (0, 0)),
            pl.BlockSpec((BT_S, DIM), lambda i: (i, 0)),
            pl.BlockSpec((1, 1, BT_S), lambda i: (i, 0, 0)),
        ],
        out_specs=pl.BlockSpec((BT_S, DIM), lambda i: (i, 0)),
        out_shape=jax.ShapeDtypeStruct((T, DIM), jnp.float32),
    )(xf, Ws1, Ws3, Ws2, yg, gate3)

    return out.reshape(shape)
